# Initial kernel scaffold; baseline (speedup 1.0000x reference)
import jax
import jax.numpy as jnp
from jax.experimental import pallas as pl


def kernel(input_ids, token_type_ids, token_table, pos_table, seg_table, ln_gamma, ln_beta):
    raise NotImplementedError("write your pallas kernel here")


# speedup vs baseline: 1.9546x; 1.9546x over previous
import functools

import jax
import jax.numpy as jnp
from jax import lax
from jax.experimental import pallas as pl
from jax.experimental.pallas import tpu as pltpu
from jax.experimental.pallas import tpu_sc as plsc

B, S = 4, 2048
HIDDEN = 768
N_TOK = B * S
NC, NS = 2, 16
NW = NC * NS
TOK_PER_W = N_TOK // NW
CHUNK = 64
N_CHUNKS = TOK_PER_W // CHUNK

_sc_mesh = plsc.VectorSubcoreMesh(
    core_axis_name="c", subcore_axis_name="s", num_cores=NC, num_subcores=NS
)


@functools.partial(
    pl.kernel,
    out_type=jax.ShapeDtypeStruct((N_TOK, HIDDEN), jnp.float32),
    mesh=_sc_mesh,
    scratch_types=[
        pltpu.VMEM((CHUNK,), jnp.int32),
        pltpu.VMEM((CHUNK, HIDDEN), jnp.float32),
        pltpu.SemaphoreType.DMA,
    ],
)
def _sc_gather(ids_hbm, table_hbm, out_hbm, idx_v, rows_v, sem):
    wid = lax.axis_index("s") * NC + lax.axis_index("c")
    base = wid * TOK_PER_W
    for c in range(N_CHUNKS):
        tok = base + c * CHUNK
        pltpu.sync_copy(ids_hbm.at[pl.ds(tok, CHUNK)], idx_v)
        pltpu.async_copy(table_hbm.at[idx_v], rows_v, sem).wait()
        pltpu.sync_copy(rows_v, out_hbm.at[pl.ds(tok, CHUNK)])


TC_BLK = 256
POS_BLKS = S // TC_BLK


def _tc_body(g_ref, pos_ref, tt_ref, seg_ref, gam_ref, bet_ref, out_ref):
    x = g_ref[...] + pos_ref[...]
    tt = tt_ref[...]
    seg = seg_ref[...]
    s0 = seg[0:1, :]
    x = x + s0 + tt * (seg[1:2, :] - s0)
    mean = jnp.mean(x, axis=-1, keepdims=True)
    xc = x - mean
    var = jnp.mean(xc * xc, axis=-1, keepdims=True)
    y = xc * lax.rsqrt(var + 1e-12)
    out_ref[...] = y * gam_ref[...] + bet_ref[...]


def _tc_ln(gathered, pos_table, ttf, seg_table, gamma2d, beta2d):
    grid = N_TOK // TC_BLK
    return pl.pallas_call(
        _tc_body,
        grid=(grid,),
        in_specs=[
            pl.BlockSpec((TC_BLK, HIDDEN), lambda i: (i, 0)),
            pl.BlockSpec((TC_BLK, HIDDEN), lambda i: (i % POS_BLKS, 0)),
            pl.BlockSpec((TC_BLK, 1), lambda i: (i, 0)),
            pl.BlockSpec((2, HIDDEN), lambda i: (0, 0)),
            pl.BlockSpec((1, HIDDEN), lambda i: (0, 0)),
            pl.BlockSpec((1, HIDDEN), lambda i: (0, 0)),
        ],
        out_specs=pl.BlockSpec((TC_BLK, HIDDEN), lambda i: (i, 0)),
        out_shape=jax.ShapeDtypeStruct((N_TOK, HIDDEN), jnp.float32),
    )(gathered, pos_table, ttf, seg_table, gamma2d, beta2d)


def kernel(input_ids, token_type_ids, token_table, pos_table, seg_table, ln_gamma, ln_beta):
    ids_flat = input_ids.reshape(-1).astype(jnp.int32)
    ttf = token_type_ids.reshape(-1, 1).astype(jnp.float32)
    gathered = _sc_gather(ids_flat, token_table)
    out = _tc_ln(
        gathered,
        pos_table,
        ttf,
        seg_table,
        ln_gamma.reshape(1, HIDDEN),
        ln_beta.reshape(1, HIDDEN),
    )
    return out.reshape(B, S, HIDDEN)


# speedup vs baseline: 2.0170x; 1.0319x over previous
import functools

import jax
import jax.numpy as jnp
from jax import lax
from jax.experimental import pallas as pl
from jax.experimental.pallas import tpu as pltpu
from jax.experimental.pallas import tpu_sc as plsc

B, S = 4, 2048
HIDDEN = 768
N_TOK = B * S
NC, NS = 2, 16
NW = NC * NS
TOK_PER_W = N_TOK // NW
CHUNK = 64
N_CHUNKS = TOK_PER_W // CHUNK
NBUF = 2

_sc_mesh = plsc.VectorSubcoreMesh(
    core_axis_name="c", subcore_axis_name="s", num_cores=NC, num_subcores=NS
)


@functools.partial(
    pl.kernel,
    out_type=jax.ShapeDtypeStruct((N_TOK, HIDDEN), jnp.float32),
    mesh=_sc_mesh,
    scratch_types=[
        pltpu.VMEM((N_CHUNKS, CHUNK), jnp.int32),
        pltpu.VMEM((NBUF, CHUNK, HIDDEN), jnp.float32),
        pltpu.SemaphoreType.DMA,
        pltpu.SemaphoreType.DMA,
    ],
)
def _sc_gather(ids_hbm, table_hbm, out_hbm, idx_v, rows_v, gsem, wsem):
    wid = lax.axis_index("s") * NC + lax.axis_index("c")
    base = wid * TOK_PER_W
    pltpu.sync_copy(ids_hbm.at[wid], idx_v)

    def gather(c, buf):
        return pltpu.make_async_copy(
            table_hbm.at[idx_v.at[c]], rows_v.at[buf], gsem
        )

    def writeback(c, buf):
        return pltpu.make_async_copy(
            rows_v.at[buf], out_hbm.at[pl.ds(base + c * CHUNK, CHUNK)], wsem
        )

    gather(0, 0).start()
    for c in range(N_CHUNKS):
        buf = c % NBUF
        gather(c, buf).wait()
        if c + 1 < N_CHUNKS:
            if c + 1 >= NBUF:
                writeback(c + 1 - NBUF, (c + 1) % NBUF).wait()
            gather(c + 1, (c + 1) % NBUF).start()
        writeback(c, buf).start()
    for c in range(max(0, N_CHUNKS - NBUF), N_CHUNKS):
        writeback(c, c % NBUF).wait()


TC_BLK = 256
POS_BLKS = S // TC_BLK


def _tc_body(g_ref, pos_ref, tt_ref, seg_ref, gam_ref, bet_ref, out_ref):
    x = g_ref[...] + pos_ref[...]
    tt = tt_ref[...]
    seg = seg_ref[...]
    s0 = seg[0:1, :]
    x = x + s0 + tt * (seg[1:2, :] - s0)
    mean = jnp.mean(x, axis=-1, keepdims=True)
    xc = x - mean
    var = jnp.mean(xc * xc, axis=-1, keepdims=True)
    y = xc * lax.rsqrt(var + 1e-12)
    out_ref[...] = y * gam_ref[...] + bet_ref[...]


def _tc_ln(gathered, pos_table, ttf, seg_table, gamma2d, beta2d):
    return pl.pallas_call(
        _tc_body,
        grid=(POS_BLKS, B),
        in_specs=[
            pl.BlockSpec((TC_BLK, HIDDEN), lambda j, b: (b * POS_BLKS + j, 0)),
            pl.BlockSpec((TC_BLK, HIDDEN), lambda j, b: (j, 0)),
            pl.BlockSpec((TC_BLK, 1), lambda j, b: (b * POS_BLKS + j, 0)),
            pl.BlockSpec((2, HIDDEN), lambda j, b: (0, 0)),
            pl.BlockSpec((1, HIDDEN), lambda j, b: (0, 0)),
            pl.BlockSpec((1, HIDDEN), lambda j, b: (0, 0)),
        ],
        out_specs=pl.BlockSpec((TC_BLK, HIDDEN), lambda j, b: (b * POS_BLKS + j, 0)),
        out_shape=jax.ShapeDtypeStruct((N_TOK, HIDDEN), jnp.float32),
    )(gathered, pos_table, ttf, seg_table, gamma2d, beta2d)


def kernel(input_ids, token_type_ids, token_table, pos_table, seg_table, ln_gamma, ln_beta):
    ids_w = input_ids.reshape(NW, N_CHUNKS, CHUNK).astype(jnp.int32)
    ttf = token_type_ids.reshape(-1, 1).astype(jnp.float32)
    gathered = _sc_gather(ids_w, token_table)
    out = _tc_ln(
        gathered,
        pos_table,
        ttf,
        seg_table,
        ln_gamma.reshape(1, HIDDEN),
        ln_beta.reshape(1, HIDDEN),
    )
    return out.reshape(B, S, HIDDEN)


# speedup vs baseline: 2.3239x; 1.1522x over previous
import functools

import jax
import jax.numpy as jnp
from jax import lax
from jax.experimental import pallas as pl
from jax.experimental.pallas import tpu as pltpu
from jax.experimental.pallas import tpu_sc as plsc

B, S = 4, 2048
HIDDEN = 768
N_TOK = B * S
NC, NS = 2, 16
NW = NC * NS
TOK_PER_W = N_TOK // NW
CHUNK = 64
N_CHUNKS = TOK_PER_W // CHUNK
NBUF = 2

_sc_mesh = plsc.VectorSubcoreMesh(
    core_axis_name="c", subcore_axis_name="s", num_cores=NC, num_subcores=NS
)


@functools.partial(
    pl.kernel,
    out_type=jax.ShapeDtypeStruct((N_TOK, HIDDEN), jnp.float32),
    mesh=_sc_mesh,
    scratch_types=[
        pltpu.VMEM((N_CHUNKS, CHUNK), jnp.int32),
        pltpu.VMEM((NBUF, CHUNK, HIDDEN), jnp.float32),
        pltpu.SemaphoreType.DMA,
        pltpu.SemaphoreType.DMA,
    ],
)
def _sc_gather(ids_hbm, table_hbm, out_hbm, idx_v, rows_v, gsem, wsem):
    wid = lax.axis_index("s") * NC + lax.axis_index("c")
    base = wid * TOK_PER_W
    pltpu.sync_copy(ids_hbm.at[wid], idx_v)

    def gather(c, buf):
        return pltpu.make_async_copy(
            table_hbm.at[idx_v.at[c]], rows_v.at[buf], gsem
        )

    def writeback(c, buf):
        return pltpu.make_async_copy(
            rows_v.at[buf], out_hbm.at[pl.ds(base + c * CHUNK, CHUNK)], wsem
        )

    gather(0, 0).start()
    for c in range(N_CHUNKS):
        buf = c % NBUF
        gather(c, buf).wait()
        if c + 1 < N_CHUNKS:
            if c + 1 >= NBUF:
                writeback(c + 1 - NBUF, (c + 1) % NBUF).wait()
            gather(c + 1, (c + 1) % NBUF).start()
        writeback(c, buf).start()
    for c in range(max(0, N_CHUNKS - NBUF), N_CHUNKS):
        writeback(c, c % NBUF).wait()


TC_BLK = 512
POS_BLKS = S // TC_BLK


def _tc_body(g_ref, pos_ref, tt_ref, seg_ref, gam_ref, bet_ref, out_ref):
    x = g_ref[...] + pos_ref[...]
    tt = tt_ref[...]
    seg = seg_ref[...]
    s0 = seg[0:1, :]
    x = x + s0 + tt * (seg[1:2, :] - s0)
    mean = jnp.mean(x, axis=-1, keepdims=True)
    xc = x - mean
    var = jnp.mean(xc * xc, axis=-1, keepdims=True)
    y = xc * lax.rsqrt(var + 1e-12)
    out_ref[...] = y * gam_ref[...] + bet_ref[...]


def _tc_ln(gathered, pos_table, ttf, seg_table, gamma2d, beta2d):
    return pl.pallas_call(
        _tc_body,
        grid=(POS_BLKS, B),
        in_specs=[
            pl.BlockSpec((TC_BLK, HIDDEN), lambda j, b: (b * POS_BLKS + j, 0)),
            pl.BlockSpec((TC_BLK, HIDDEN), lambda j, b: (j, 0)),
            pl.BlockSpec((TC_BLK, 1), lambda j, b: (b * POS_BLKS + j, 0)),
            pl.BlockSpec((2, HIDDEN), lambda j, b: (0, 0)),
            pl.BlockSpec((1, HIDDEN), lambda j, b: (0, 0)),
            pl.BlockSpec((1, HIDDEN), lambda j, b: (0, 0)),
        ],
        out_specs=pl.BlockSpec((TC_BLK, HIDDEN), lambda j, b: (b * POS_BLKS + j, 0)),
        out_shape=jax.ShapeDtypeStruct((N_TOK, HIDDEN), jnp.float32),
    )(gathered, pos_table, ttf, seg_table, gamma2d, beta2d)


def kernel(input_ids, token_type_ids, token_table, pos_table, seg_table, ln_gamma, ln_beta):
    ids_w = input_ids.reshape(NW, N_CHUNKS, CHUNK).astype(jnp.int32)
    ttf = token_type_ids.reshape(-1, 1).astype(jnp.float32)
    gathered = _sc_gather(ids_w, token_table)
    out = _tc_ln(
        gathered,
        pos_table,
        ttf,
        seg_table,
        ln_gamma.reshape(1, HIDDEN),
        ln_beta.reshape(1, HIDDEN),
    )
    return out.reshape(B, S, HIDDEN)


# speedup vs baseline: 2.4514x; 1.0549x over previous
import functools

import jax
import jax.numpy as jnp
from jax import lax
from jax.experimental import pallas as pl
from jax.experimental.pallas import tpu as pltpu
from jax.experimental.pallas import tpu_sc as plsc

B, S = 4, 2048
HIDDEN = 768
N_TOK = B * S
NC, NS = 2, 16
NW = NC * NS
TOK_PER_W = N_TOK // NW
CHUNK = 64
N_CHUNKS = TOK_PER_W // CHUNK
NBUF = 2

_sc_mesh = plsc.VectorSubcoreMesh(
    core_axis_name="c", subcore_axis_name="s", num_cores=NC, num_subcores=NS
)


@functools.partial(
    pl.kernel,
    out_type=jax.ShapeDtypeStruct((N_TOK, HIDDEN), jnp.float32),
    mesh=_sc_mesh,
    scratch_types=[
        pltpu.VMEM((N_CHUNKS, CHUNK), jnp.int32),
        pltpu.VMEM((NBUF, CHUNK, HIDDEN), jnp.float32),
        pltpu.SemaphoreType.DMA,
        pltpu.SemaphoreType.DMA,
    ],
)
def _sc_gather(ids_hbm, table_hbm, out_hbm, idx_v, rows_v, gsem, wsem):
    wid = lax.axis_index("s") * NC + lax.axis_index("c")
    base = wid * TOK_PER_W
    pltpu.sync_copy(ids_hbm.at[wid], idx_v)

    def gather(c, buf):
        return pltpu.make_async_copy(
            table_hbm.at[idx_v.at[c]], rows_v.at[buf], gsem
        )

    def writeback(c, buf):
        return pltpu.make_async_copy(
            rows_v.at[buf], out_hbm.at[pl.ds(base + c * CHUNK, CHUNK)], wsem
        )

    gather(0, 0).start()
    for c in range(N_CHUNKS):
        buf = c % NBUF
        gather(c, buf).wait()
        if c + 1 < N_CHUNKS:
            if c + 1 >= NBUF:
                writeback(c + 1 - NBUF, (c + 1) % NBUF).wait()
            gather(c + 1, (c + 1) % NBUF).start()
        writeback(c, buf).start()
    for c in range(max(0, N_CHUNKS - NBUF), N_CHUNKS):
        writeback(c, c % NBUF).wait()


TC_BLK = 1024
POS_BLKS = S // TC_BLK


def _tc_body(g_ref, pos_ref, tt_ref, seg_ref, gam_ref, bet_ref, out_ref):
    x = g_ref[...] + pos_ref[...]
    tt = tt_ref[...]
    seg = seg_ref[...]
    s0 = seg[0:1, :]
    x = x + s0 + tt * (seg[1:2, :] - s0)
    mean = jnp.mean(x, axis=-1, keepdims=True)
    xc = x - mean
    var = jnp.mean(xc * xc, axis=-1, keepdims=True)
    y = xc * lax.rsqrt(var + 1e-12)
    out_ref[...] = y * gam_ref[...] + bet_ref[...]


def _tc_ln(gathered, pos_table, ttf, seg_table, gamma2d, beta2d):
    return pl.pallas_call(
        _tc_body,
        grid=(POS_BLKS, B),
        in_specs=[
            pl.BlockSpec((TC_BLK, HIDDEN), lambda j, b: (b * POS_BLKS + j, 0)),
            pl.BlockSpec((TC_BLK, HIDDEN), lambda j, b: (j, 0)),
            pl.BlockSpec((TC_BLK, 1), lambda j, b: (b * POS_BLKS + j, 0)),
            pl.BlockSpec((2, HIDDEN), lambda j, b: (0, 0)),
            pl.BlockSpec((1, HIDDEN), lambda j, b: (0, 0)),
            pl.BlockSpec((1, HIDDEN), lambda j, b: (0, 0)),
        ],
        out_specs=pl.BlockSpec((TC_BLK, HIDDEN), lambda j, b: (b * POS_BLKS + j, 0)),
        out_shape=jax.ShapeDtypeStruct((N_TOK, HIDDEN), jnp.float32),
    )(gathered, pos_table, ttf, seg_table, gamma2d, beta2d)


def kernel(input_ids, token_type_ids, token_table, pos_table, seg_table, ln_gamma, ln_beta):
    ids_w = input_ids.reshape(NW, N_CHUNKS, CHUNK).astype(jnp.int32)
    ttf = token_type_ids.reshape(-1, 1).astype(jnp.float32)
    gathered = _sc_gather(ids_w, token_table)
    out = _tc_ln(
        gathered,
        pos_table,
        ttf,
        seg_table,
        ln_gamma.reshape(1, HIDDEN),
        ln_beta.reshape(1, HIDDEN),
    )
    return out.reshape(B, S, HIDDEN)


# speedup vs baseline: 2.4640x; 1.0051x over previous
import functools

import jax
import jax.numpy as jnp
from jax import lax
from jax.experimental import pallas as pl
from jax.experimental.pallas import tpu as pltpu
from jax.experimental.pallas import tpu_sc as plsc

B, S = 4, 2048
HIDDEN = 768
N_TOK = B * S
NC, NS = 2, 16
NW = NC * NS
TOK_PER_W = N_TOK // NW
CHUNK = 64
N_CHUNKS = TOK_PER_W // CHUNK
NBUF = 2

_sc_mesh = plsc.VectorSubcoreMesh(
    core_axis_name="c", subcore_axis_name="s", num_cores=NC, num_subcores=NS
)


@functools.partial(
    pl.kernel,
    out_type=jax.ShapeDtypeStruct((N_TOK, HIDDEN), jnp.float32),
    mesh=_sc_mesh,
    scratch_types=[
        pltpu.VMEM((N_CHUNKS, CHUNK), jnp.int32),
        pltpu.VMEM((NBUF, CHUNK, HIDDEN), jnp.float32),
        pltpu.SemaphoreType.DMA,
        pltpu.SemaphoreType.DMA,
    ],
)
def _sc_gather(ids_hbm, table_hbm, out_hbm, idx_v, rows_v, gsem, wsem):
    wid = lax.axis_index("s") * NC + lax.axis_index("c")
    base = wid * TOK_PER_W
    pltpu.sync_copy(ids_hbm.at[wid], idx_v)

    def gather(c, buf):
        return pltpu.make_async_copy(
            table_hbm.at[idx_v.at[c]], rows_v.at[buf], gsem
        )

    def writeback(c, buf):
        return pltpu.make_async_copy(
            rows_v.at[buf], out_hbm.at[pl.ds(base + c * CHUNK, CHUNK)], wsem
        )

    gather(0, 0).start()
    for c in range(N_CHUNKS):
        buf = c % NBUF
        gather(c, buf).wait()
        if c + 1 < N_CHUNKS:
            if c + 1 >= NBUF:
                writeback(c + 1 - NBUF, (c + 1) % NBUF).wait()
            gather(c + 1, (c + 1) % NBUF).start()
        writeback(c, buf).start()
    for c in range(max(0, N_CHUNKS - NBUF), N_CHUNKS):
        writeback(c, c % NBUF).wait()


TC_BLK = 2048
POS_BLKS = S // TC_BLK


def _tc_body(g_ref, pos_ref, tt_ref, seg_ref, gam_ref, bet_ref, out_ref):
    x = g_ref[...] + pos_ref[...]
    tt = tt_ref[...]
    seg = seg_ref[...]
    s0 = seg[0:1, :]
    x = x + s0 + tt * (seg[1:2, :] - s0)
    mean = jnp.mean(x, axis=-1, keepdims=True)
    xc = x - mean
    var = jnp.mean(xc * xc, axis=-1, keepdims=True)
    y = xc * lax.rsqrt(var + 1e-12)
    out_ref[...] = y * gam_ref[...] + bet_ref[...]


def _tc_ln(gathered, pos_table, ttf, seg_table, gamma2d, beta2d):
    return pl.pallas_call(
        _tc_body,
        grid=(POS_BLKS, B),
        in_specs=[
            pl.BlockSpec((TC_BLK, HIDDEN), lambda j, b: (b * POS_BLKS + j, 0)),
            pl.BlockSpec((TC_BLK, HIDDEN), lambda j, b: (j, 0)),
            pl.BlockSpec((TC_BLK, 1), lambda j, b: (b * POS_BLKS + j, 0)),
            pl.BlockSpec((2, HIDDEN), lambda j, b: (0, 0)),
            pl.BlockSpec((1, HIDDEN), lambda j, b: (0, 0)),
            pl.BlockSpec((1, HIDDEN), lambda j, b: (0, 0)),
        ],
        out_specs=pl.BlockSpec((TC_BLK, HIDDEN), lambda j, b: (b * POS_BLKS + j, 0)),
        out_shape=jax.ShapeDtypeStruct((N_TOK, HIDDEN), jnp.float32),
    )(gathered, pos_table, ttf, seg_table, gamma2d, beta2d)


def kernel(input_ids, token_type_ids, token_table, pos_table, seg_table, ln_gamma, ln_beta):
    ids_w = input_ids.reshape(NW, N_CHUNKS, CHUNK).astype(jnp.int32)
    ttf = token_type_ids.reshape(-1, 1).astype(jnp.float32)
    gathered = _sc_gather(ids_w, token_table)
    out = _tc_ln(
        gathered,
        pos_table,
        ttf,
        seg_table,
        ln_gamma.reshape(1, HIDDEN),
        ln_beta.reshape(1, HIDDEN),
    )
    return out.reshape(B, S, HIDDEN)
